# baseline (speedup 1.0000x reference)
import functools

import jax
import jax.numpy as jnp
from jax import lax
from jax.experimental import pallas as pl
from jax.experimental.pallas import tpu as pltpu
from jax.experimental.pallas import tpu_sc as plsc

BATCH = 16384
HIST = 50
EMBED = 32
NC = 2
NS = 16
NW = NC * NS
BT = BATCH // 128
UNITS = HIST * BT
PER_W = UNITS // NW
U = 5
NG = PER_W // U
L = 16


def _build():
    mesh = plsc.VectorSubcoreMesh(core_axis_name="c", subcore_axis_name="s")

    @functools.partial(
        pl.kernel,
        mesh=mesh,
        out_type=jax.ShapeDtypeStruct((HIST, EMBED // 8, BT, 8, 128), jnp.float32),
        scratch_types=[
            pltpu.VMEM((PER_W, 128), jnp.int32),
            pltpu.VMEM((U * 128, EMBED), jnp.float32),
            pltpu.VMEM((U * 128, EMBED), jnp.float32),
            pltpu.VMEM((U, EMBED, 128), jnp.float32),
            pltpu.VMEM((U, EMBED, 128), jnp.float32),
            pltpu.SemaphoreType.DMA,
            pltpu.SemaphoreType.DMA,
            pltpu.SemaphoreType.DMA,
            pltpu.SemaphoreType.DMA,
        ],
        compiler_params=pltpu.CompilerParams(
            use_tc_tiling_on_sc=False, needs_layout_passes=False),
    )
    def gather_kernel(idx_hbm, table_hbm, out_hbm, idx_v, rows_a, rows_b,
                      tb_a, tb_b, gsem_a, gsem_b, wsem_a, wsem_b):
        wid = lax.axis_index("s") * NC + lax.axis_index("c")
        ubase = wid * PER_W
        pltpu.sync_copy(idx_hbm.at[pl.ds(ubase, PER_W)], idx_v)

        rows_bufs = (rows_a, rows_b)
        tb_bufs = (tb_a, tb_b)
        gsems = (gsem_a, gsem_b)
        wsems = (wsem_a, wsem_b)

        def gather_copies(g, par):
            return [
                pltpu.make_async_copy(
                    table_hbm.at[idx_v.at[g * U + j]],
                    rows_bufs[par].at[pl.ds(j * 128, 128)],
                    gsems[par],
                )
                for j in range(U)
            ]

        def wb_copies(g, par):
            cps = []
            for j in range(U):
                u = ubase + g * U + j
                h = u // BT
                bt = lax.rem(u, BT)
                tb = tb_bufs[par]
                for et in range(EMBED // 8):
                    cps.append(
                        pltpu.make_async_copy(
                            tb.at[j, pl.ds(et * 8, 8)],
                            out_hbm.at[h, et, bt],
                            wsems[par],
                        )
                    )
            return cps

        def transpose_group(par):
            rows = rows_bufs[par]
            tb = tb_bufs[par]
            iot = lax.iota(jnp.int32, L)
            for j in range(U):
                ridxs = [j * 128 + rb * L + iot for rb in range(128 // L)]

                @pl.loop(0, EMBED, unroll=4)
                def e_loop(e):
                    ce = jnp.broadcast_to(e, (L,))
                    for rb in range(128 // L):
                        x = plsc.load_gather(rows, [ridxs[rb], ce])
                        tb[j, e, pl.ds(rb * L, L)] = x

        for cp in gather_copies(0, 0):
            cp.start()

        @pl.loop(0, NG, step=2)
        def pair_body(g0):
            for p in range(2):
                g = g0 + p

                @pl.when(g + 1 < NG)
                def _fire_next():
                    for cp in gather_copies(g + 1, 1 - p):
                        cp.start()

                for cp in gather_copies(g, p):
                    cp.wait()

                @pl.when(g >= 2)
                def _drain_wb():
                    for cp in wb_copies(g - 2, p):
                        cp.wait()

                transpose_group(p)
                for cp in wb_copies(g, p):
                    cp.start()

        for cp in wb_copies(NG - 2, 0):
            cp.wait()
        for cp in wb_copies(NG - 1, 1):
            cp.wait()

    return gather_kernel


_GATHER = _build()


def kernel(inputs, table):
    idx_t = jnp.transpose(inputs.astype(jnp.int32)).reshape(UNITS, 128)
    packed = _GATHER(idx_t, table)
    return packed.transpose(2, 4, 0, 1, 3).reshape(BATCH, HIST, EMBED)


# speedup vs baseline: 1.5455x; 1.5455x over previous
import functools

import jax
import jax.numpy as jnp
from jax import lax
from jax.experimental import pallas as pl
from jax.experimental.pallas import tpu as pltpu
from jax.experimental.pallas import tpu_sc as plsc

BATCH = 16384
HIST = 50
EMBED = 32
NC = 2
NS = 16
NW = NC * NS
BT = BATCH // 128
UNITS = HIST * BT
PER_W = UNITS // NW
U = 5
NG = PER_W // U
L = 16


def _build():
    mesh = plsc.VectorSubcoreMesh(core_axis_name="c", subcore_axis_name="s")

    @functools.partial(
        pl.kernel,
        mesh=mesh,
        out_type=jax.ShapeDtypeStruct((HIST, EMBED // 8, BT, 8, 128), jnp.float32),
        scratch_types=[
            pltpu.VMEM((PER_W, 128), jnp.int32),
            pltpu.VMEM((U * 128, EMBED), jnp.float32),
            pltpu.VMEM((U * 128, EMBED), jnp.float32),
            pltpu.VMEM((U, EMBED, 128), jnp.float32),
            pltpu.VMEM((U, EMBED, 128), jnp.float32),
            pltpu.SemaphoreType.DMA,
            pltpu.SemaphoreType.DMA,
            pltpu.SemaphoreType.DMA,
            pltpu.SemaphoreType.DMA,
        ],
        compiler_params=pltpu.CompilerParams(
            use_tc_tiling_on_sc=False, needs_layout_passes=False),
    )
    def gather_kernel(idx_hbm, table_hbm, out_hbm, idx_v, rows_a, rows_b,
                      tb_a, tb_b, gsem_a, gsem_b, wsem_a, wsem_b):
        wid = lax.axis_index("s") * NC + lax.axis_index("c")
        ubase = wid * PER_W
        pltpu.sync_copy(idx_hbm.at[pl.ds(ubase, PER_W)], idx_v)

        rows_bufs = (rows_a, rows_b)
        tb_bufs = (tb_a, tb_b)
        gsems = (gsem_a, gsem_b)
        wsems = (wsem_a, wsem_b)

        def gather_copies(g, par):
            return [
                pltpu.make_async_copy(
                    table_hbm.at[idx_v.at[g * U + j]],
                    rows_bufs[par].at[pl.ds(j * 128, 128)],
                    gsems[par],
                )
                for j in range(U)
            ]

        def wb_copies(g, par):
            cps = []
            for j in range(U):
                u = ubase + g * U + j
                h = u // BT
                bt = lax.rem(u, BT)
                tb = tb_bufs[par]
                for et in range(EMBED // 8):
                    cps.append(
                        pltpu.make_async_copy(
                            tb.at[j, pl.ds(et * 8, 8)],
                            out_hbm.at[h, et, bt],
                            wsems[par],
                        )
                    )
            return cps

        def transpose_group(par):
            rows = rows_bufs[par]
            tb = tb_bufs[par]
            iot = lax.iota(jnp.int32, L)
            for j in range(U):
                bidxs = [rb * L + iot for rb in range(128 // L)]
                ridxs = [j * 128 + rb * L + iot for rb in range(128 // L)]
                cj = jnp.broadcast_to(jnp.int32(j), (L,))

                @pl.loop(0, EMBED, unroll=4)
                def e_loop(e):
                    cidx = lax.rem(e + iot, jnp.broadcast_to(jnp.int32(EMBED), (L,)))
                    for rb in range(128 // L):
                        x = plsc.load_gather(rows, [ridxs[rb], cidx])
                        plsc.store_scatter(tb, [cj, cidx, bidxs[rb]], x)

        for cp in gather_copies(0, 0):
            cp.start()

        @pl.loop(0, NG, step=2)
        def pair_body(g0):
            for p in range(2):
                g = g0 + p

                @pl.when(g + 1 < NG)
                def _fire_next():
                    for cp in gather_copies(g + 1, 1 - p):
                        cp.start()

                for cp in gather_copies(g, p):
                    cp.wait()

                @pl.when(g >= 2)
                def _drain_wb():
                    for cp in wb_copies(g - 2, p):
                        cp.wait()

                transpose_group(p)
                for cp in wb_copies(g, p):
                    cp.start()

        for cp in wb_copies(NG - 2, 0):
            cp.wait()
        for cp in wb_copies(NG - 1, 1):
            cp.wait()

    return gather_kernel


_GATHER = _build()


def kernel(inputs, table):
    idx_t = jnp.transpose(inputs.astype(jnp.int32)).reshape(UNITS, 128)
    packed = _GATHER(idx_t, table)
    return packed.transpose(2, 4, 0, 1, 3).reshape(BATCH, HIST, EMBED)


# speedup vs baseline: 2.1792x; 1.4100x over previous
import functools

import jax
import jax.numpy as jnp
from jax import lax
from jax.experimental import pallas as pl
from jax.experimental.pallas import tpu as pltpu
from jax.experimental.pallas import tpu_sc as plsc

BATCH = 16384
HIST = 50
EMBED = 32
VOCAB = 1000000
NC = 2
NS = 16
NW = NC * NS
L = 16

VT = 7813
VOCAB_PAD = VT * 128
VT_MAIN = 7808
VT_PER_W = VT_MAIN // NW

BT = BATCH // 128
UNITS = HIST * BT
PER_W = UNITS // NW
U = 5
NG = PER_W // U


def _build_detile():
    mesh = plsc.VectorSubcoreMesh(core_axis_name="c", subcore_axis_name="s")

    @functools.partial(
        pl.kernel,
        mesh=mesh,
        out_type=jax.ShapeDtypeStruct((VOCAB_PAD * EMBED // 128, 128), jnp.float32),
        scratch_types=[
            pltpu.VMEM((EMBED, 128), jnp.float32),
            pltpu.VMEM((EMBED, 128), jnp.float32),
            pltpu.VMEM((EMBED, 128), jnp.float32),
            pltpu.VMEM((EMBED, 128), jnp.float32),
            pltpu.SemaphoreType.DMA,
            pltpu.SemaphoreType.DMA,
            pltpu.SemaphoreType.DMA,
            pltpu.SemaphoreType.DMA,
        ],
        compiler_params=pltpu.CompilerParams(
            use_tc_tiling_on_sc=True, needs_layout_passes=False),
    )
    def detile_kernel(tab_t_hbm, tail_hbm, out_hbm, tiles_a, tiles_b,
                      rows_a, rows_b, tsem_a, tsem_b, wsem_a, wsem_b):
        wid = lax.axis_index("s") * NC + lax.axis_index("c")
        vt0 = wid * VT_PER_W
        tiles_bufs = (tiles_a, tiles_b)
        rows_bufs = (rows_a, rows_b)
        tsems = (tsem_a, tsem_b)
        wsems = (wsem_a, wsem_b)
        iot = lax.iota(jnp.int32, L)
        vlv = [vb * L + iot for vb in range(8)]
        rv = [lax.shift_right_logical(iot, 2) + 4 * vb for vb in range(8)]
        base32 = lax.bitwise_and(iot, 3) * EMBED

        def in_copies(vt, par):
            return [
                pltpu.make_async_copy(
                    tab_t_hbm.at[pl.ds(et * 8, 8), pl.ds(vt * 128, 128)],
                    tiles_bufs[par].at[pl.ds(et * 8, 8)],
                    tsems[par],
                )
                for et in range(4)
            ]

        def out_copy(vt, par):
            return pltpu.make_async_copy(
                rows_bufs[par],
                out_hbm.at[pl.ds(vt * EMBED, EMBED)],
                wsems[par],
            )

        def transpose_tile(par):
            tiles = tiles_bufs[par]
            rows = rows_bufs[par]

            @pl.loop(0, EMBED)
            def d_loop(d):
                cidx = lax.bitwise_and(d + iot, EMBED - 1)
                cv = base32 + cidx
                for vb in range(8):
                    x = plsc.load_gather(tiles, [cidx, vlv[vb]])
                    plsc.store_scatter(rows, [rv[vb], cv], x)

        for cp in in_copies(vt0, 0):
            cp.start()

        @pl.loop(0, VT_PER_W, step=2)
        def pair_body(g0):
            for p in range(2):
                g = g0 + p
                vt = vt0 + g

                @pl.when(g + 1 < VT_PER_W)
                def _fire_next():
                    for cp in in_copies(vt + 1, 1 - p):
                        cp.start()

                for cp in in_copies(vt, p):
                    cp.wait()

                @pl.when(g >= 2)
                def _drain_wb():
                    out_copy(vt - 2, p).wait()

                transpose_tile(p)
                out_copy(vt, p).start()

        out_copy(vt0 + VT_PER_W - 2, 0).wait()
        out_copy(vt0 + VT_PER_W - 1, 1).wait()

        @pl.when(wid < 4)
        def _tail_full():
            vt = VT_MAIN + wid
            for et in range(4):
                pltpu.sync_copy(
                    tab_t_hbm.at[pl.ds(et * 8, 8), pl.ds(vt * 128, 128)],
                    tiles_a.at[pl.ds(et * 8, 8)],
                )
            transpose_tile(0)
            pltpu.sync_copy(rows_a, out_hbm.at[pl.ds(vt * EMBED, EMBED)])

        @pl.when(wid == 4)
        def _tail_part():
            pltpu.sync_copy(tail_hbm, rows_a.at[pl.ds(0, 16)])
            pltpu.sync_copy(
                rows_a.at[pl.ds(0, 16)],
                out_hbm.at[pl.ds((VT - 1) * EMBED, 16)],
            )

    return detile_kernel


def _build_gather():
    mesh = plsc.VectorSubcoreMesh(core_axis_name="c", subcore_axis_name="s")

    @functools.partial(
        pl.kernel,
        mesh=mesh,
        out_type=jax.ShapeDtypeStruct((HIST, EMBED // 8, BT, 8, 128), jnp.float32),
        scratch_types=[
            pltpu.VMEM((PER_W, 128), jnp.int32),
            pltpu.VMEM((U * 128, EMBED), jnp.float32),
            pltpu.VMEM((U * 128, EMBED), jnp.float32),
            pltpu.VMEM((U, EMBED, 128), jnp.float32),
            pltpu.VMEM((U, EMBED, 128), jnp.float32),
            pltpu.SemaphoreType.DMA,
            pltpu.SemaphoreType.DMA,
            pltpu.SemaphoreType.DMA,
            pltpu.SemaphoreType.DMA,
        ],
        compiler_params=pltpu.CompilerParams(
            use_tc_tiling_on_sc=False, needs_layout_passes=False),
    )
    def gather_kernel(idx_hbm, table_hbm, out_hbm, idx_v, rows_a, rows_b,
                      tb_a, tb_b, gsem_a, gsem_b, wsem_a, wsem_b):
        wid = lax.axis_index("s") * NC + lax.axis_index("c")
        ubase = wid * PER_W
        pltpu.sync_copy(idx_hbm.at[pl.ds(ubase, PER_W)], idx_v)

        rows_bufs = (rows_a, rows_b)
        tb_bufs = (tb_a, tb_b)
        gsems = (gsem_a, gsem_b)
        wsems = (wsem_a, wsem_b)

        def gather_copies(g, par):
            return [
                pltpu.make_async_copy(
                    table_hbm.at[idx_v.at[g * U + j]],
                    rows_bufs[par].at[pl.ds(j * 128, 128)],
                    gsems[par],
                )
                for j in range(U)
            ]

        def wb_copies(g, par):
            cps = []
            for j in range(U):
                u = ubase + g * U + j
                h = u // BT
                bt = lax.rem(u, BT)
                tb = tb_bufs[par]
                for et in range(EMBED // 8):
                    cps.append(
                        pltpu.make_async_copy(
                            tb.at[j, pl.ds(et * 8, 8)],
                            out_hbm.at[h, et, bt],
                            wsems[par],
                        )
                    )
            return cps

        def transpose_group(par):
            rows = rows_bufs[par]
            tb = tb_bufs[par]
            iot = lax.iota(jnp.int32, L)
            for j in range(U):
                bidxs = [rb * L + iot for rb in range(128 // L)]
                ridxs = [j * 128 + rb * L + iot for rb in range(128 // L)]
                cj = jnp.broadcast_to(jnp.int32(j), (L,))

                @pl.loop(0, EMBED, unroll=4)
                def e_loop(e):
                    cidx = lax.rem(e + iot, jnp.broadcast_to(jnp.int32(EMBED), (L,)))
                    for rb in range(128 // L):
                        x = plsc.load_gather(rows, [ridxs[rb], cidx])
                        plsc.store_scatter(tb, [cj, cidx, bidxs[rb]], x)

        for cp in gather_copies(0, 0):
            cp.start()

        @pl.loop(0, NG, step=2)
        def pair_body(g0):
            for p in range(2):
                g = g0 + p

                @pl.when(g + 1 < NG)
                def _fire_next():
                    for cp in gather_copies(g + 1, 1 - p):
                        cp.start()

                for cp in gather_copies(g, p):
                    cp.wait()

                @pl.when(g >= 2)
                def _drain_wb():
                    for cp in wb_copies(g - 2, p):
                        cp.wait()

                transpose_group(p)
                for cp in wb_copies(g, p):
                    cp.start()

        for cp in wb_copies(NG - 2, 0):
            cp.wait()
        for cp in wb_copies(NG - 1, 1):
            cp.wait()

    return gather_kernel


_DETILE = _build_detile()
_GATHER = _build_gather()


def kernel(inputs, table):
    tail = table[VT_MAIN * 128 + 512:].reshape(16, 128)
    table_lin = _DETILE(jnp.transpose(table), tail).reshape(VOCAB_PAD, EMBED)
    idx_t = jnp.transpose(inputs.astype(jnp.int32)).reshape(UNITS, 128)
    packed = _GATHER(idx_t, table_lin)
    return packed.transpose(2, 4, 0, 1, 3).reshape(BATCH, HIST, EMBED)


# speedup vs baseline: 2.2139x; 1.0159x over previous
import functools

import jax
import jax.numpy as jnp
from jax import lax
from jax.experimental import pallas as pl
from jax.experimental.pallas import tpu as pltpu
from jax.experimental.pallas import tpu_sc as plsc

BATCH = 16384
HIST = 50
EMBED = 32
VOCAB = 1000000
NC = 2
NS = 16
NW = NC * NS
L = 16

VT = 7813
VOCAB_PAD = VT * 128
VT_MAIN = 7808
VT_PER_W = VT_MAIN // NW

BT = BATCH // 128
UNITS = HIST * BT
PER_W = UNITS // NW
U = 5
NG = PER_W // U


def _build_detile():
    mesh = plsc.VectorSubcoreMesh(core_axis_name="c", subcore_axis_name="s")

    @functools.partial(
        pl.kernel,
        mesh=mesh,
        out_type=jax.ShapeDtypeStruct((VOCAB_PAD * EMBED // 128, 128), jnp.float32),
        scratch_types=[
            pltpu.VMEM((EMBED, 128), jnp.float32),
            pltpu.VMEM((EMBED, 128), jnp.float32),
            pltpu.VMEM((EMBED, 128), jnp.float32),
            pltpu.VMEM((EMBED, 128), jnp.float32),
            pltpu.SemaphoreType.DMA,
            pltpu.SemaphoreType.DMA,
            pltpu.SemaphoreType.DMA,
            pltpu.SemaphoreType.DMA,
        ],
        compiler_params=pltpu.CompilerParams(
            use_tc_tiling_on_sc=True, needs_layout_passes=False),
    )
    def detile_kernel(tab_t_hbm, tail_hbm, out_hbm, tiles_a, tiles_b,
                      rows_a, rows_b, tsem_a, tsem_b, wsem_a, wsem_b):
        wid = lax.axis_index("s") * NC + lax.axis_index("c")
        vt0 = wid * VT_PER_W
        tiles_bufs = (tiles_a, tiles_b)
        rows_bufs = (rows_a, rows_b)
        tsems = (tsem_a, tsem_b)
        wsems = (wsem_a, wsem_b)
        iot = lax.iota(jnp.int32, L)
        vlv = [vb * L + iot for vb in range(8)]
        rv = [lax.shift_right_logical(iot, 2) + 4 * vb for vb in range(8)]
        base32 = lax.bitwise_and(iot, 3) * EMBED

        def in_copies(vt, par):
            return [
                pltpu.make_async_copy(
                    tab_t_hbm.at[pl.ds(et * 8, 8), pl.ds(vt * 128, 128)],
                    tiles_bufs[par].at[pl.ds(et * 8, 8)],
                    tsems[par],
                )
                for et in range(4)
            ]

        def out_copy(vt, par):
            return pltpu.make_async_copy(
                rows_bufs[par],
                out_hbm.at[pl.ds(vt * EMBED, EMBED)],
                wsems[par],
            )

        def transpose_tile(par):
            tiles = tiles_bufs[par]
            rows = rows_bufs[par]

            @pl.loop(0, EMBED, unroll=4)
            def d_loop(d):
                cidx = lax.bitwise_and(d + iot, EMBED - 1)
                cv = base32 + cidx
                for vb in range(8):
                    x = plsc.load_gather(tiles, [cidx, vlv[vb]])
                    plsc.store_scatter(rows, [rv[vb], cv], x)

        for cp in in_copies(vt0, 0):
            cp.start()

        @pl.loop(0, VT_PER_W, step=2)
        def pair_body(g0):
            for p in range(2):
                g = g0 + p
                vt = vt0 + g

                @pl.when(g + 1 < VT_PER_W)
                def _fire_next():
                    for cp in in_copies(vt + 1, 1 - p):
                        cp.start()

                for cp in in_copies(vt, p):
                    cp.wait()

                @pl.when(g >= 2)
                def _drain_wb():
                    out_copy(vt - 2, p).wait()

                transpose_tile(p)
                out_copy(vt, p).start()

        out_copy(vt0 + VT_PER_W - 2, 0).wait()
        out_copy(vt0 + VT_PER_W - 1, 1).wait()

        @pl.when(wid < 4)
        def _tail_full():
            vt = VT_MAIN + wid
            for et in range(4):
                pltpu.sync_copy(
                    tab_t_hbm.at[pl.ds(et * 8, 8), pl.ds(vt * 128, 128)],
                    tiles_a.at[pl.ds(et * 8, 8)],
                )
            transpose_tile(0)
            pltpu.sync_copy(rows_a, out_hbm.at[pl.ds(vt * EMBED, EMBED)])

        @pl.when(wid == 4)
        def _tail_part():
            pltpu.sync_copy(tail_hbm, rows_a.at[pl.ds(0, 16)])
            pltpu.sync_copy(
                rows_a.at[pl.ds(0, 16)],
                out_hbm.at[pl.ds((VT - 1) * EMBED, 16)],
            )

    return detile_kernel


def _build_gather():
    mesh = plsc.VectorSubcoreMesh(core_axis_name="c", subcore_axis_name="s")

    @functools.partial(
        pl.kernel,
        mesh=mesh,
        out_type=jax.ShapeDtypeStruct((HIST, EMBED // 8, BT, 8, 128), jnp.float32),
        scratch_types=[
            pltpu.VMEM((PER_W, 128), jnp.int32),
            pltpu.VMEM((U * 128, EMBED), jnp.float32),
            pltpu.VMEM((U * 128, EMBED), jnp.float32),
            pltpu.VMEM((U, EMBED, 128), jnp.float32),
            pltpu.VMEM((U, EMBED, 128), jnp.float32),
            pltpu.SemaphoreType.DMA,
            pltpu.SemaphoreType.DMA,
            pltpu.SemaphoreType.DMA,
            pltpu.SemaphoreType.DMA,
        ],
        compiler_params=pltpu.CompilerParams(
            use_tc_tiling_on_sc=False, needs_layout_passes=False),
    )
    def gather_kernel(idx_hbm, table_hbm, out_hbm, idx_v, rows_a, rows_b,
                      tb_a, tb_b, gsem_a, gsem_b, wsem_a, wsem_b):
        wid = lax.axis_index("s") * NC + lax.axis_index("c")
        ubase = wid * PER_W
        pltpu.sync_copy(idx_hbm.at[pl.ds(ubase, PER_W)], idx_v)

        rows_bufs = (rows_a, rows_b)
        tb_bufs = (tb_a, tb_b)
        gsems = (gsem_a, gsem_b)
        wsems = (wsem_a, wsem_b)

        def gather_copies(g, par):
            return [
                pltpu.make_async_copy(
                    table_hbm.at[idx_v.at[g * U + j]],
                    rows_bufs[par].at[pl.ds(j * 128, 128)],
                    gsems[par],
                )
                for j in range(U)
            ]

        def wb_copies(g, par):
            cps = []
            for j in range(U):
                u = ubase + g * U + j
                h = u // BT
                bt = lax.rem(u, BT)
                tb = tb_bufs[par]
                for et in range(EMBED // 8):
                    cps.append(
                        pltpu.make_async_copy(
                            tb.at[j, pl.ds(et * 8, 8)],
                            out_hbm.at[h, et, bt],
                            wsems[par],
                        )
                    )
            return cps

        def transpose_group(par):
            rows = rows_bufs[par]
            tb = tb_bufs[par]
            iot = lax.iota(jnp.int32, L)
            for j in range(U):
                bidxs = [rb * L + iot for rb in range(128 // L)]
                ridxs = [j * 128 + rb * L + iot for rb in range(128 // L)]
                cj = jnp.broadcast_to(jnp.int32(j), (L,))

                @pl.loop(0, EMBED, unroll=4)
                def e_loop(e):
                    cidx = lax.bitwise_and(e + iot, EMBED - 1)
                    for rb in range(128 // L):
                        x = plsc.load_gather(rows, [ridxs[rb], cidx])
                        plsc.store_scatter(tb, [cj, cidx, bidxs[rb]], x)

        for cp in gather_copies(0, 0):
            cp.start()

        @pl.loop(0, NG, step=2)
        def pair_body(g0):
            for p in range(2):
                g = g0 + p

                @pl.when(g + 1 < NG)
                def _fire_next():
                    for cp in gather_copies(g + 1, 1 - p):
                        cp.start()

                for cp in gather_copies(g, p):
                    cp.wait()

                @pl.when(g >= 2)
                def _drain_wb():
                    for cp in wb_copies(g - 2, p):
                        cp.wait()

                transpose_group(p)
                for cp in wb_copies(g, p):
                    cp.start()

        for cp in wb_copies(NG - 2, 0):
            cp.wait()
        for cp in wb_copies(NG - 1, 1):
            cp.wait()

    return gather_kernel


_DETILE = _build_detile()
_GATHER = _build_gather()


def kernel(inputs, table):
    tail = table[VT_MAIN * 128 + 512:].reshape(16, 128)
    table_lin = _DETILE(jnp.transpose(table), tail).reshape(VOCAB_PAD, EMBED)
    idx_t = jnp.transpose(inputs.astype(jnp.int32)).reshape(UNITS, 128)
    packed = _GATHER(idx_t, table_lin)
    return packed.transpose(2, 4, 0, 1, 3).reshape(BATCH, HIST, EMBED)


# speedup vs baseline: 4.0973x; 1.8507x over previous
import functools

import jax
import jax.numpy as jnp
from jax import lax
from jax.experimental import pallas as pl
from jax.experimental.pallas import tpu as pltpu
from jax.experimental.pallas import tpu_sc as plsc

BATCH = 16384
HIST = 50
EMBED = 32
VOCAB = 1000000
NC = 2
NS = 16
NW = NC * NS
L = 16

VT = 7813
VOCAB_PAD = VT * 128
VT_MAIN = 7808
VT_PER_W = VT_MAIN // NW

BT = BATCH // 128
UNITS = HIST * BT
PER_W = UNITS // NW
U = 5
NG = PER_W // U


def _build_detile():
    mesh = plsc.VectorSubcoreMesh(core_axis_name="c", subcore_axis_name="s")

    @functools.partial(
        pl.kernel,
        mesh=mesh,
        out_type=jax.ShapeDtypeStruct((VOCAB_PAD * EMBED // 128, 128), jnp.float32),
        scratch_types=[
            pltpu.VMEM((EMBED, 128), jnp.float32),
            pltpu.VMEM((EMBED, 128), jnp.float32),
            pltpu.VMEM((EMBED, 128), jnp.float32),
            pltpu.VMEM((EMBED, 128), jnp.float32),
            pltpu.SemaphoreType.DMA,
            pltpu.SemaphoreType.DMA,
            pltpu.SemaphoreType.DMA,
            pltpu.SemaphoreType.DMA,
        ],
        compiler_params=pltpu.CompilerParams(
            use_tc_tiling_on_sc=True, needs_layout_passes=False),
    )
    def detile_kernel(tab_t_hbm, tail_hbm, out_hbm, tiles_a, tiles_b,
                      rows_a, rows_b, tsem_a, tsem_b, wsem_a, wsem_b):
        wid = lax.axis_index("s") * NC + lax.axis_index("c")
        vt0 = wid * VT_PER_W
        tiles_bufs = (tiles_a, tiles_b)
        rows_bufs = (rows_a, rows_b)
        tsems = (tsem_a, tsem_b)
        wsems = (wsem_a, wsem_b)
        iot = lax.iota(jnp.int32, L)
        vlv = [vb * L + iot for vb in range(8)]
        rv = [lax.shift_right_logical(iot, 2) + 4 * vb for vb in range(8)]
        base32 = lax.bitwise_and(iot, 3) * EMBED

        def in_copies(vt, par):
            return [
                pltpu.make_async_copy(
                    tab_t_hbm.at[pl.ds(et * 8, 8), pl.ds(vt * 128, 128)],
                    tiles_bufs[par].at[pl.ds(et * 8, 8)],
                    tsems[par],
                )
                for et in range(4)
            ]

        def out_copy(vt, par):
            return pltpu.make_async_copy(
                rows_bufs[par],
                out_hbm.at[pl.ds(vt * EMBED, EMBED)],
                wsems[par],
            )

        def transpose_tile(par):
            tiles = tiles_bufs[par]
            rows = rows_bufs[par]

            @pl.loop(0, EMBED, unroll=4)
            def d_loop(d):
                cidx = lax.bitwise_and(d + iot, EMBED - 1)
                cv = base32 + cidx
                xs = [plsc.load_gather(tiles, [cidx, vlv[vb]]) for vb in range(8)]
                for vb in range(8):
                    plsc.store_scatter(rows, [rv[vb], cv], xs[vb])

        for cp in in_copies(vt0, 0):
            cp.start()

        @pl.loop(0, VT_PER_W, step=2)
        def pair_body(g0):
            for p in range(2):
                g = g0 + p
                vt = vt0 + g

                @pl.when(g + 1 < VT_PER_W)
                def _fire_next():
                    for cp in in_copies(vt + 1, 1 - p):
                        cp.start()

                for cp in in_copies(vt, p):
                    cp.wait()

                @pl.when(g >= 2)
                def _drain_wb():
                    out_copy(vt - 2, p).wait()

                transpose_tile(p)
                out_copy(vt, p).start()

        out_copy(vt0 + VT_PER_W - 2, 0).wait()
        out_copy(vt0 + VT_PER_W - 1, 1).wait()

        @pl.when(wid < 4)
        def _tail_full():
            vt = VT_MAIN + wid
            for et in range(4):
                pltpu.sync_copy(
                    tab_t_hbm.at[pl.ds(et * 8, 8), pl.ds(vt * 128, 128)],
                    tiles_a.at[pl.ds(et * 8, 8)],
                )
            transpose_tile(0)
            pltpu.sync_copy(rows_a, out_hbm.at[pl.ds(vt * EMBED, EMBED)])

        @pl.when(wid == 4)
        def _tail_part():
            pltpu.sync_copy(tail_hbm, rows_a.at[pl.ds(0, 16)])
            pltpu.sync_copy(
                rows_a.at[pl.ds(0, 16)],
                out_hbm.at[pl.ds((VT - 1) * EMBED, 16)],
            )

    return detile_kernel


def _build_gather():
    mesh = plsc.VectorSubcoreMesh(core_axis_name="c", subcore_axis_name="s")

    @functools.partial(
        pl.kernel,
        mesh=mesh,
        out_type=jax.ShapeDtypeStruct((HIST, EMBED // 8, BT, 8, 128), jnp.float32),
        scratch_types=[
            pltpu.VMEM((PER_W, 128), jnp.int32),
            pltpu.VMEM((U * 128, EMBED), jnp.float32),
            pltpu.VMEM((U * 128, EMBED), jnp.float32),
            pltpu.VMEM((U, EMBED, 128), jnp.float32),
            pltpu.VMEM((U, EMBED, 128), jnp.float32),
            pltpu.SemaphoreType.DMA,
            pltpu.SemaphoreType.DMA,
            pltpu.SemaphoreType.DMA,
            pltpu.SemaphoreType.DMA,
        ],
        compiler_params=pltpu.CompilerParams(
            use_tc_tiling_on_sc=False, needs_layout_passes=False),
    )
    def gather_kernel(idx_hbm, table_hbm, out_hbm, idx_v, rows_a, rows_b,
                      tb_a, tb_b, gsem_a, gsem_b, wsem_a, wsem_b):
        wid = lax.axis_index("s") * NC + lax.axis_index("c")
        ubase = wid * PER_W
        pltpu.sync_copy(idx_hbm.at[pl.ds(ubase, PER_W)], idx_v)

        rows_bufs = (rows_a, rows_b)
        tb_bufs = (tb_a, tb_b)
        gsems = (gsem_a, gsem_b)
        wsems = (wsem_a, wsem_b)

        def gather_copies(g, par):
            return [
                pltpu.make_async_copy(
                    table_hbm.at[idx_v.at[g * U + j]],
                    rows_bufs[par].at[pl.ds(j * 128, 128)],
                    gsems[par],
                )
                for j in range(U)
            ]

        def wb_copies(g, par):
            cps = []
            for j in range(U):
                u = ubase + g * U + j
                h = u // BT
                bt = lax.rem(u, BT)
                tb = tb_bufs[par]
                for et in range(EMBED // 8):
                    cps.append(
                        pltpu.make_async_copy(
                            tb.at[j, pl.ds(et * 8, 8)],
                            out_hbm.at[h, et, bt],
                            wsems[par],
                        )
                    )
            return cps

        def transpose_group(par):
            rows = rows_bufs[par]
            tb = tb_bufs[par]
            iot = lax.iota(jnp.int32, L)
            for j in range(U):
                bidxs = [rb * L + iot for rb in range(128 // L)]
                ridxs = [j * 128 + rb * L + iot for rb in range(128 // L)]
                cj = jnp.broadcast_to(jnp.int32(j), (L,))

                @pl.loop(0, EMBED, unroll=4)
                def e_loop(e):
                    cidx = lax.bitwise_and(e + iot, EMBED - 1)
                    xs = [plsc.load_gather(rows, [ridxs[rb], cidx])
                          for rb in range(128 // L)]
                    for rb in range(128 // L):
                        plsc.store_scatter(tb, [cj, cidx, bidxs[rb]], xs[rb])

        for cp in gather_copies(0, 0):
            cp.start()

        @pl.loop(0, NG, step=2)
        def pair_body(g0):
            for p in range(2):
                g = g0 + p

                @pl.when(g + 1 < NG)
                def _fire_next():
                    for cp in gather_copies(g + 1, 1 - p):
                        cp.start()

                for cp in gather_copies(g, p):
                    cp.wait()

                @pl.when(g >= 2)
                def _drain_wb():
                    for cp in wb_copies(g - 2, p):
                        cp.wait()

                transpose_group(p)
                for cp in wb_copies(g, p):
                    cp.start()

        for cp in wb_copies(NG - 2, 0):
            cp.wait()
        for cp in wb_copies(NG - 1, 1):
            cp.wait()

    return gather_kernel


_DETILE = _build_detile()
_GATHER = _build_gather()


def kernel(inputs, table):
    tail = table[VT_MAIN * 128 + 512:].reshape(16, 128)
    table_lin = _DETILE(jnp.transpose(table), tail).reshape(VOCAB_PAD, EMBED)
    idx_t = jnp.transpose(inputs.astype(jnp.int32)).reshape(UNITS, 128)
    packed = _GATHER(idx_t, table_lin)
    return packed.transpose(2, 4, 0, 1, 3).reshape(BATCH, HIST, EMBED)


# speedup vs baseline: 4.8173x; 1.1757x over previous
import functools

import jax
import jax.numpy as jnp
from jax import lax
from jax.experimental import pallas as pl
from jax.experimental.pallas import tpu as pltpu
from jax.experimental.pallas import tpu_sc as plsc

BATCH = 16384
HIST = 50
EMBED = 32
VOCAB = 1000000
NC = 2
NS = 16
NW = NC * NS
L = 16

VT = 7813
VOCAB_PAD = VT * 128
VT_MAIN = 7808
VT_PER_W = VT_MAIN // NW

BT = BATCH // 128
UNITS = HIST * BT
PER_W = UNITS // NW
U = 5
NG = PER_W // U


def _build_detile():
    mesh = plsc.VectorSubcoreMesh(core_axis_name="c", subcore_axis_name="s")

    @functools.partial(
        pl.kernel,
        mesh=mesh,
        out_type=jax.ShapeDtypeStruct((VOCAB_PAD * EMBED // 128, 128), jnp.float32),
        scratch_types=[
            pltpu.VMEM((EMBED, 256), jnp.float32),
            pltpu.VMEM((EMBED, 256), jnp.float32),
            pltpu.VMEM((2 * EMBED, 128), jnp.float32),
            pltpu.VMEM((2 * EMBED, 128), jnp.float32),
            pltpu.SemaphoreType.DMA,
            pltpu.SemaphoreType.DMA,
            pltpu.SemaphoreType.DMA,
            pltpu.SemaphoreType.DMA,
        ],
        compiler_params=pltpu.CompilerParams(
            use_tc_tiling_on_sc=True, needs_layout_passes=False),
    )
    def detile_kernel(tab_t_hbm, tail_hbm, out_hbm, tiles_a, tiles_b,
                      rows_a, rows_b, tsem_a, tsem_b, wsem_a, wsem_b):
        wid = lax.axis_index("s") * NC + lax.axis_index("c")
        vt0 = wid * VT_PER_W
        tiles_bufs = (tiles_a, tiles_b)
        rows_bufs = (rows_a, rows_b)
        tsems = (tsem_a, tsem_b)
        wsems = (wsem_a, wsem_b)
        iot = lax.iota(jnp.int32, L)
        vlv = [[t * 128 + vb * L + iot for vb in range(8)] for t in range(2)]
        rv = [[t * EMBED + lax.shift_right_logical(iot, 2) + 4 * vb
               for vb in range(8)] for t in range(2)]
        base32 = lax.bitwise_and(iot, 3) * EMBED

        NP = VT_PER_W // 2

        def in_copies(vp, par):
            return [
                pltpu.make_async_copy(
                    tab_t_hbm.at[pl.ds(et * 8, 8), pl.ds((vp * 2 + t) * 128, 128)],
                    tiles_bufs[par].at[pl.ds(et * 8, 8), pl.ds(t * 128, 128)],
                    tsems[par],
                )
                for t in range(2)
                for et in range(4)
            ]

        def out_copy(vp, par):
            return pltpu.make_async_copy(
                rows_bufs[par],
                out_hbm.at[pl.ds(vp * 2 * EMBED, 2 * EMBED)],
                wsems[par],
            )

        def transpose_tiles(par, ts):
            tiles = tiles_bufs[par]
            rows = rows_bufs[par]

            @pl.loop(0, EMBED, unroll=4)
            def d_loop(d):
                cidx = lax.bitwise_and(d + iot, EMBED - 1)
                cv = base32 + cidx
                xs = [plsc.load_gather(tiles, [cidx, vlv[t][vb]])
                      for t in ts for vb in range(8)]
                for i, (t, vb) in enumerate([(t, vb) for t in ts
                                             for vb in range(8)]):
                    plsc.store_scatter(rows, [rv[t][vb], cv], xs[i])

        for cp in in_copies(vp0 := wid * NP, 0):
            cp.start()

        @pl.loop(0, NP, step=2)
        def pair_body(g0):
            for p in range(2):
                g = g0 + p
                vp = vp0 + g

                @pl.when(g + 1 < NP)
                def _fire_next():
                    for cp in in_copies(vp + 1, 1 - p):
                        cp.start()

                for cp in in_copies(vp, p):
                    cp.wait()

                @pl.when(g >= 2)
                def _drain_wb():
                    out_copy(vp - 2, p).wait()

                transpose_tiles(p, (0, 1))
                out_copy(vp, p).start()

        out_copy(vp0 + NP - 2, 0).wait()
        out_copy(vp0 + NP - 1, 1).wait()

        @pl.when(wid < 4)
        def _tail_full():
            vt = VT_MAIN + wid
            for et in range(4):
                pltpu.sync_copy(
                    tab_t_hbm.at[pl.ds(et * 8, 8), pl.ds(vt * 128, 128)],
                    tiles_a.at[pl.ds(et * 8, 8), pl.ds(0, 128)],
                )
            transpose_tiles(0, (0,))
            pltpu.sync_copy(rows_a.at[pl.ds(0, EMBED)],
                            out_hbm.at[pl.ds(vt * EMBED, EMBED)])

        @pl.when(wid == 4)
        def _tail_part():
            pltpu.sync_copy(tail_hbm, rows_a.at[pl.ds(0, 16)])
            pltpu.sync_copy(
                rows_a.at[pl.ds(0, 16)],
                out_hbm.at[pl.ds((VT - 1) * EMBED, 16)],
            )

    return detile_kernel


def _build_gather():
    mesh = plsc.VectorSubcoreMesh(core_axis_name="c", subcore_axis_name="s")

    @functools.partial(
        pl.kernel,
        mesh=mesh,
        out_type=jax.ShapeDtypeStruct((HIST, EMBED // 8, BT, 8, 128), jnp.float32),
        scratch_types=[
            pltpu.VMEM((PER_W, 128), jnp.int32),
            pltpu.VMEM((U * 128, EMBED), jnp.float32),
            pltpu.VMEM((U * 128, EMBED), jnp.float32),
            pltpu.VMEM((U, EMBED, 128), jnp.float32),
            pltpu.VMEM((U, EMBED, 128), jnp.float32),
            pltpu.SemaphoreType.DMA,
            pltpu.SemaphoreType.DMA,
            pltpu.SemaphoreType.DMA,
            pltpu.SemaphoreType.DMA,
        ],
        compiler_params=pltpu.CompilerParams(
            use_tc_tiling_on_sc=False, needs_layout_passes=False),
    )
    def gather_kernel(idx_hbm, table_hbm, out_hbm, idx_v, rows_a, rows_b,
                      tb_a, tb_b, gsem_a, gsem_b, wsem_a, wsem_b):
        wid = lax.axis_index("s") * NC + lax.axis_index("c")
        ubase = wid * PER_W
        pltpu.sync_copy(idx_hbm.at[pl.ds(ubase, PER_W)], idx_v)

        rows_bufs = (rows_a, rows_b)
        tb_bufs = (tb_a, tb_b)
        gsems = (gsem_a, gsem_b)
        wsems = (wsem_a, wsem_b)

        def gather_copies(g, par):
            return [
                pltpu.make_async_copy(
                    table_hbm.at[idx_v.at[g * U + j]],
                    rows_bufs[par].at[pl.ds(j * 128, 128)],
                    gsems[par],
                )
                for j in range(U)
            ]

        def wb_copies(g, par):
            cps = []
            for j in range(U):
                u = ubase + g * U + j
                h = u // BT
                bt = lax.rem(u, BT)
                tb = tb_bufs[par]
                for et in range(EMBED // 8):
                    cps.append(
                        pltpu.make_async_copy(
                            tb.at[j, pl.ds(et * 8, 8)],
                            out_hbm.at[h, et, bt],
                            wsems[par],
                        )
                    )
            return cps

        def transpose_group(par):
            rows = rows_bufs[par]
            tb = tb_bufs[par]
            iot = lax.iota(jnp.int32, L)
            for j in range(U):
                bidxs = [rb * L + iot for rb in range(128 // L)]
                ridxs = [j * 128 + rb * L + iot for rb in range(128 // L)]
                cj = jnp.broadcast_to(jnp.int32(j), (L,))

                @pl.loop(0, EMBED, unroll=4)
                def e_loop(e):
                    cidx = lax.bitwise_and(e + iot, EMBED - 1)
                    xs = [plsc.load_gather(rows, [ridxs[rb], cidx])
                          for rb in range(128 // L)]
                    for rb in range(128 // L):
                        plsc.store_scatter(tb, [cj, cidx, bidxs[rb]], xs[rb])

        for cp in gather_copies(0, 0):
            cp.start()

        @pl.loop(0, NG, step=2)
        def pair_body(g0):
            for p in range(2):
                g = g0 + p

                @pl.when(g + 1 < NG)
                def _fire_next():
                    for cp in gather_copies(g + 1, 1 - p):
                        cp.start()

                for cp in gather_copies(g, p):
                    cp.wait()

                @pl.when(g >= 2)
                def _drain_wb():
                    for cp in wb_copies(g - 2, p):
                        cp.wait()

                transpose_group(p)
                for cp in wb_copies(g, p):
                    cp.start()

        for cp in wb_copies(NG - 2, 0):
            cp.wait()
        for cp in wb_copies(NG - 1, 1):
            cp.wait()

    return gather_kernel


_DETILE = _build_detile()
_GATHER = _build_gather()


def kernel(inputs, table):
    tail = table[VT_MAIN * 128 + 512:].reshape(16, 128)
    table_lin = _DETILE(jnp.transpose(table), tail).reshape(VOCAB_PAD, EMBED)
    idx_t = jnp.transpose(inputs.astype(jnp.int32)).reshape(UNITS, 128)
    packed = _GATHER(idx_t, table_lin)
    return packed.transpose(2, 4, 0, 1, 3).reshape(BATCH, HIST, EMBED)
